# baseline (speedup 1.0000x reference)
import functools

import jax
import jax.numpy as jnp
from jax import lax
from jax.experimental import pallas as pl
from jax.experimental.pallas import tpu as pltpu
from jax.experimental.pallas import tpu_sc as plsc

N = 10000
E = 320000
D = 128
B = 20
PER = 500
K = 250
NPAD = 10240
GP = 512
KP = 256
NC = 2
NS = 16
NW = NC * NS
CH = 128
NCHUNK = E // CH
EPC = 2560
EP = EPC * CH
CPT = EPC // NW
SLICE = NPAD // NS
ROWS_PT = (B * KP) // NW
NEG = -3.0e38

_MESH = dict(core_axis_name="c", subcore_axis_name="s", num_cores=NC,
             num_subcores=NS)


def _fill(ref, n, value):
    def body(i, carry):
        ref[pl.ds(i * 16, 16)] = jnp.full((16,), value, ref.dtype)
        return carry
    lax.fori_loop(0, n // 16, body, 0)


def _fill2d(ref, rows, cols, value):
    def body(i, carry):
        r = i // (cols // 16)
        c = i % (cols // 16)
        ref[r, pl.ds(c * 16, 16)] = jnp.full((16,), value, ref.dtype)
        return carry
    lax.fori_loop(0, rows * (cols // 16), body, 0)


def _deg_body(epk_hbm, out_hbm, sh_do, sh_di, idx4, ones_v, buf_v, l0, l1):
    cid = lax.axis_index("c")
    tid = lax.axis_index("s")
    wid = cid * NS + tid
    lsem = (l0, l1)
    _fill(ones_v, CH, 1.0)
    _fill(buf_v, SLICE, 0.0)
    pltpu.sync_copy(buf_v, sh_do.at[pl.ds(tid * SLICE, SLICE)])
    pltpu.sync_copy(buf_v, sh_di.at[pl.ds(tid * SLICE, SLICE)])
    plsc.subcore_barrier()
    pltpu.async_copy(epk_hbm.at[wid], idx4.at[0], lsem[0])

    def pair(j, carry):
        for p in (0, 1):
            pltpu.make_async_copy(epk_hbm.at[0], idx4.at[p], lsem[p]).wait()

            def _load():
                c_next = wid + (j * 2 + p + 1) * NW
                pltpu.async_copy(epk_hbm.at[c_next], idx4.at[1 - p],
                                 lsem[1 - p])
            if p == 0:
                _load()
            else:
                pl.when(j < (CPT // 2) - 1)(_load)
            pltpu.sync_copy(ones_v, sh_do.at[idx4.at[p, 0]], add=True)
            pltpu.sync_copy(ones_v, sh_di.at[idx4.at[p, 1]], add=True)
        return carry
    lax.fori_loop(0, CPT // 2, pair, 0)
    plsc.subcore_barrier()
    pltpu.sync_copy(sh_do.at[pl.ds(tid * SLICE, SLICE)], buf_v)
    pltpu.sync_copy(buf_v, out_hbm.at[cid, 0, pl.ds(tid * SLICE, SLICE)])
    pltpu.sync_copy(sh_di.at[pl.ds(tid * SLICE, SLICE)], buf_v)
    pltpu.sync_copy(buf_v, out_hbm.at[cid, 1, pl.ds(tid * SLICE, SLICE)])


_deg_call = pl.kernel(
    _deg_body,
    out_type=jax.ShapeDtypeStruct((NC, 2, NPAD), jnp.float32),
    mesh=plsc.VectorSubcoreMesh(**_MESH),
    scratch_types=[
        pltpu.VMEM_SHARED((NPAD,), jnp.float32),
        pltpu.VMEM_SHARED((NPAD,), jnp.float32),
        pltpu.VMEM((2, 2, CH), jnp.int32),
        pltpu.VMEM((CH,), jnp.float32),
        pltpu.VMEM((SLICE,), jnp.float32),
    ] + [pltpu.SemaphoreType.DMA] * 2,
)


def _agg_body(epk_hbm, h_hbm, out_hbm, sh_acc, idx4, rows2, l0, l1):
    cid = lax.axis_index("c")
    tid = lax.axis_index("s")
    wid = cid * NS + tid
    lsem = (l0, l1)
    _fill2d(rows2.at[0], CH, D, 0.0)
    for j in range(SLICE // CH):
        pltpu.sync_copy(rows2.at[0], sh_acc.at[pl.ds(tid * SLICE + j * CH, CH)])
    plsc.subcore_barrier()
    pltpu.async_copy(epk_hbm.at[wid], idx4.at[0], lsem[0])

    def pair(j, carry):
        for p in (0, 1):
            pltpu.make_async_copy(epk_hbm.at[0], idx4.at[p], lsem[p]).wait()

            def _load():
                c_next = wid + (j * 2 + p + 1) * NW
                pltpu.async_copy(epk_hbm.at[c_next], idx4.at[1 - p],
                                 lsem[1 - p])
            if p == 0:
                _load()
            else:
                pl.when(j < (CPT // 2) - 1)(_load)
            pltpu.sync_copy(h_hbm.at[idx4.at[p, 0]], rows2.at[0])
            pltpu.sync_copy(rows2.at[0], sh_acc.at[idx4.at[p, 1]], add=True)
        return carry
    lax.fori_loop(0, CPT // 2, pair, 0)
    plsc.subcore_barrier()
    for j in range(SLICE // CH):
        pltpu.sync_copy(sh_acc.at[pl.ds(tid * SLICE + j * CH, CH)],
                        rows2.at[0])
        pltpu.sync_copy(
            rows2.at[0], out_hbm.at[cid, pl.ds(tid * SLICE + j * CH, CH)])


_agg_call = pl.kernel(
    _agg_body,
    out_type=jax.ShapeDtypeStruct((NC, NPAD, D), jnp.float32),
    mesh=plsc.VectorSubcoreMesh(**_MESH),
    scratch_types=[
        pltpu.VMEM_SHARED((NPAD, D), jnp.float32),
        pltpu.VMEM((2, 2, CH), jnp.int32),
        pltpu.VMEM((1, CH, D), jnp.float32),
    ] + [pltpu.SemaphoreType.DMA] * 2,
)


def _gat_body(feat_hbm, idx_hbm, out_hbm, idx_a, idx_b, rows_v):
    cid = lax.axis_index("c")
    tid = lax.axis_index("s")
    wid = cid * NS + tid
    base = wid * ROWS_PT
    half = ROWS_PT // 2
    pltpu.sync_copy(idx_hbm.at[pl.ds(base, half)], idx_a)
    pltpu.sync_copy(idx_hbm.at[pl.ds(base + half, half)], idx_b)
    pltpu.sync_copy(feat_hbm.at[idx_a], rows_v.at[pl.ds(0, half)])
    pltpu.sync_copy(feat_hbm.at[idx_b], rows_v.at[pl.ds(half, half)])
    pltpu.sync_copy(rows_v, out_hbm.at[pl.ds(base, ROWS_PT)])


_gat_call = pl.kernel(
    _gat_body,
    out_type=jax.ShapeDtypeStruct((B * KP, D), jnp.float32),
    mesh=plsc.VectorSubcoreMesh(**_MESH),
    scratch_types=[
        pltpu.VMEM((ROWS_PT // 2,), jnp.int32),
        pltpu.VMEM((ROWS_PT // 2,), jnp.int32),
        pltpu.VMEM((ROWS_PT, D), jnp.float32),
    ],
)


def _prep_body(f_ref, do0, do1, di0, di1, h_ref, ni_ref):
    deg_out = do0[...] + do1[...]
    deg_in = di0[...] + di1[...]
    norm_out = lax.rsqrt(jnp.maximum(deg_out, 1.0))
    h_ref[...] = f_ref[...] * norm_out
    ni_ref[...] = lax.rsqrt(jnp.maximum(deg_in, 1.0))


_prep_call = pl.pallas_call(
    _prep_body,
    out_shape=(
        jax.ShapeDtypeStruct((NPAD, D), jnp.float32),
        jax.ShapeDtypeStruct((NPAD, 1), jnp.float32),
    ),
)


def _score_body(a0_ref, a1_ref, ni_ref, w_ref, s_ref):
    agg = (a0_ref[...] + a1_ref[...]) * ni_ref[...]
    s_ref[...] = jnp.dot(agg, w_ref[...], preferred_element_type=jnp.float32)


_score_call = pl.pallas_call(
    _score_body,
    out_shape=jax.ShapeDtypeStruct((NPAD, 1), jnp.float32),
)


def _topk_body(sg_ref, b_ref, perm_ref, scale_ref):
    g = pl.program_id(0)
    s = sg_ref[...].reshape(1, GP) + b_ref[0, 0]
    lane = lax.broadcasted_iota(jnp.int32, (1, GP), 1)
    sp = jnp.where(lane < PER, s, NEG)
    sb = jnp.broadcast_to(sp, (GP, GP))
    ir = lax.broadcasted_iota(jnp.int32, (GP, GP), 0)
    ic = lax.broadcasted_iota(jnp.int32, (GP, GP), 1)
    eye = ir == ic
    s_col = jnp.sum(jnp.where(eye, sb, 0.0), axis=1, keepdims=True)
    term = (sb > s_col) | ((sb == s_col) & (ic < ir))
    cnt = jnp.sum(term.astype(jnp.float32), axis=1, keepdims=True)
    onehot = cnt == ic.astype(jnp.float32)
    sel = jnp.sum(jnp.where(onehot, ir.astype(jnp.float32), 0.0), axis=0,
                  keepdims=True)
    scol_b = jnp.broadcast_to(s_col, (GP, GP))
    ssort = jnp.sum(jnp.where(onehot, scol_b, 0.0), axis=0, keepdims=True)
    perm_ref[...] = (sel.astype(jnp.int32) + g * PER).reshape(1, 1, GP)
    scale_ref[...] = jnp.tanh(ssort).reshape(1, 1, GP)


_topk_call = pl.pallas_call(
    _topk_body,
    grid=(B,),
    in_specs=[
        pl.BlockSpec((1, 1, GP), lambda g: (g, 0, 0)),
        pl.BlockSpec((1, 1), lambda g: (0, 0)),
    ],
    out_specs=(
        pl.BlockSpec((1, 1, GP), lambda g: (g, 0, 0)),
        pl.BlockSpec((1, 1, GP), lambda g: (g, 0, 0)),
    ),
    out_shape=(
        jax.ShapeDtypeStruct((B, 1, GP), jnp.int32),
        jax.ShapeDtypeStruct((B, 1, GP), jnp.float32),
    ),
)


def _scale_body(rows_ref, sc_ref, out_ref):
    out_ref[...] = rows_ref[...] * sc_ref[...]


_scale_call = pl.pallas_call(
    _scale_body,
    out_shape=jax.ShapeDtypeStruct((B * KP, D), jnp.float32),
)


def kernel(feature, W, b, edge_index, batch_num_nodes):
    src = edge_index[0]
    dst = edge_index[1]
    fill = jnp.full((EP - E,), NPAD - 1, jnp.int32)
    epk = jnp.stack([jnp.concatenate([src, fill]).reshape(EPC, CH),
                     jnp.concatenate([dst, fill]).reshape(EPC, CH)], axis=1)

    degp = _deg_call(epk)
    fpad = jnp.concatenate(
        [feature, jnp.zeros((NPAD - N, D), jnp.float32)], axis=0)
    h, norm_in = _prep_call(
        fpad,
        degp[0, 0].reshape(NPAD, 1), degp[1, 0].reshape(NPAD, 1),
        degp[0, 1].reshape(NPAD, 1), degp[1, 1].reshape(NPAD, 1))
    accp = _agg_call(epk, h)
    score = _score_call(accp[0], accp[1], norm_in, W)

    zpad = jnp.full((B, GP - PER), NEG, jnp.float32)
    sg = jnp.concatenate([score[:N, 0].reshape(B, PER), zpad], axis=1)
    perm, scale = _topk_call(sg.reshape(B, 1, GP), b.reshape(1, 1))
    perm = perm.reshape(B, GP)
    scale = scale.reshape(B, GP)

    idx = perm[:, :KP].reshape(B * KP)
    rows = _gat_call(feature, idx)
    feat = _scale_call(rows, scale[:, :KP].reshape(B * KP, 1))

    feat_out = feat.reshape(B, KP, D)[:, :K].reshape(B * K, D)
    perm_sel = idx.reshape(B, KP)[:, :K].reshape(B * K)
    return feat_out, perm_sel


# speedup vs baseline: 1.3759x; 1.3759x over previous
import functools

import jax
import jax.numpy as jnp
from jax import lax
from jax.experimental import pallas as pl
from jax.experimental.pallas import tpu as pltpu
from jax.experimental.pallas import tpu_sc as plsc

N = 10000
E = 320000
D = 128
B = 20
PER = 500
K = 250
NPAD = 10240
GP = 512
KP = 256
NC = 2
NS = 16
NW = NC * NS
CH = 128
NCHUNK = E // CH
EPC = 2560
EP = EPC * CH
CPT = EPC // NW
SLICE = NPAD // NS
ROWS_PT = (B * KP) // NW
NEG = -3.0e38

_MESH = dict(core_axis_name="c", subcore_axis_name="s", num_cores=NC,
             num_subcores=NS)


def _fill(ref, n, value):
    def body(i, carry):
        ref[pl.ds(i * 16, 16)] = jnp.full((16,), value, ref.dtype)
        return carry
    lax.fori_loop(0, n // 16, body, 0)


def _fill2d(ref, rows, cols, value):
    def body(i, carry):
        r = i // (cols // 16)
        c = i % (cols // 16)
        ref[r, pl.ds(c * 16, 16)] = jnp.full((16,), value, ref.dtype)
        return carry
    lax.fori_loop(0, rows * (cols // 16), body, 0)


def _deg_body(src_hbm, dst_hbm, out_hbm, sh_do, sh_di, idx_v, ones_v, buf_v):
    cid = lax.axis_index("c")
    tid = lax.axis_index("s")
    wid = cid * NS + tid
    _fill(ones_v, CH, 1.0)
    _fill(buf_v, SLICE, 0.0)
    pltpu.sync_copy(buf_v, sh_do.at[pl.ds(tid * SLICE, SLICE)])
    pltpu.sync_copy(buf_v, sh_di.at[pl.ds(tid * SLICE, SLICE)])
    plsc.subcore_barrier()
    nloc = NCHUNK // NW + jnp.where(wid < NCHUNK % NW, 1, 0)

    def step(i, carry):
        base = (wid + i * NW) * CH
        pltpu.sync_copy(src_hbm.at[pl.ds(base, CH)], idx_v)
        pltpu.sync_copy(ones_v, sh_do.at[idx_v], add=True)
        pltpu.sync_copy(dst_hbm.at[pl.ds(base, CH)], idx_v)
        pltpu.sync_copy(ones_v, sh_di.at[idx_v], add=True)
        return carry
    lax.fori_loop(0, nloc, step, 0)
    plsc.subcore_barrier()
    pltpu.sync_copy(sh_do.at[pl.ds(tid * SLICE, SLICE)], buf_v)
    pltpu.sync_copy(buf_v, out_hbm.at[cid, 0, pl.ds(tid * SLICE, SLICE)])
    pltpu.sync_copy(sh_di.at[pl.ds(tid * SLICE, SLICE)], buf_v)
    pltpu.sync_copy(buf_v, out_hbm.at[cid, 1, pl.ds(tid * SLICE, SLICE)])


_deg_call = pl.kernel(
    _deg_body,
    out_type=jax.ShapeDtypeStruct((NC, 2, NPAD), jnp.float32),
    mesh=plsc.VectorSubcoreMesh(**_MESH),
    scratch_types=[
        pltpu.VMEM_SHARED((NPAD,), jnp.float32),
        pltpu.VMEM_SHARED((NPAD,), jnp.float32),
        pltpu.VMEM((CH,), jnp.int32),
        pltpu.VMEM((CH,), jnp.float32),
        pltpu.VMEM((SLICE,), jnp.float32),
    ],
)


def _agg_body(src_hbm, dst_hbm, h_hbm, out_hbm, sh_acc, idxs_v, idxd_v,
              rows_v):
    cid = lax.axis_index("c")
    tid = lax.axis_index("s")
    wid = cid * NS + tid
    _fill2d(rows_v, CH, D, 0.0)
    for j in range(SLICE // CH):
        pltpu.sync_copy(rows_v, sh_acc.at[pl.ds(tid * SLICE + j * CH, CH)])
    plsc.subcore_barrier()
    nloc = NCHUNK // NW + jnp.where(wid < NCHUNK % NW, 1, 0)

    def step(i, carry):
        base = (wid + i * NW) * CH
        pltpu.sync_copy(src_hbm.at[pl.ds(base, CH)], idxs_v)
        pltpu.sync_copy(dst_hbm.at[pl.ds(base, CH)], idxd_v)
        pltpu.sync_copy(h_hbm.at[idxs_v], rows_v)
        pltpu.sync_copy(rows_v, sh_acc.at[idxd_v], add=True)
        return carry
    lax.fori_loop(0, nloc, step, 0)
    plsc.subcore_barrier()
    for j in range(SLICE // CH):
        pltpu.sync_copy(sh_acc.at[pl.ds(tid * SLICE + j * CH, CH)], rows_v)
        pltpu.sync_copy(
            rows_v, out_hbm.at[cid, pl.ds(tid * SLICE + j * CH, CH)])


_agg_call = pl.kernel(
    _agg_body,
    out_type=jax.ShapeDtypeStruct((NC, NPAD, D), jnp.float32),
    mesh=plsc.VectorSubcoreMesh(**_MESH),
    scratch_types=[
        pltpu.VMEM_SHARED((NPAD, D), jnp.float32),
        pltpu.VMEM((CH,), jnp.int32),
        pltpu.VMEM((CH,), jnp.int32),
        pltpu.VMEM((CH, D), jnp.float32),
    ],
)


def _gat_body(feat_hbm, idx_hbm, out_hbm, idx_a, idx_b, rows_v):
    cid = lax.axis_index("c")
    tid = lax.axis_index("s")
    wid = cid * NS + tid
    base = wid * ROWS_PT
    half = ROWS_PT // 2
    pltpu.sync_copy(idx_hbm.at[pl.ds(base, half)], idx_a)
    pltpu.sync_copy(idx_hbm.at[pl.ds(base + half, half)], idx_b)
    pltpu.sync_copy(feat_hbm.at[idx_a], rows_v.at[pl.ds(0, half)])
    pltpu.sync_copy(feat_hbm.at[idx_b], rows_v.at[pl.ds(half, half)])
    pltpu.sync_copy(rows_v, out_hbm.at[pl.ds(base, ROWS_PT)])


_gat_call = pl.kernel(
    _gat_body,
    out_type=jax.ShapeDtypeStruct((B * KP, D), jnp.float32),
    mesh=plsc.VectorSubcoreMesh(**_MESH),
    scratch_types=[
        pltpu.VMEM((ROWS_PT // 2,), jnp.int32),
        pltpu.VMEM((ROWS_PT // 2,), jnp.int32),
        pltpu.VMEM((ROWS_PT, D), jnp.float32),
    ],
)


def _prep_body(f_ref, do0, do1, di0, di1, h_ref, ni_ref):
    deg_out = do0[...] + do1[...]
    deg_in = di0[...] + di1[...]
    norm_out = lax.rsqrt(jnp.maximum(deg_out, 1.0))
    h_ref[...] = f_ref[...] * norm_out
    ni_ref[...] = lax.rsqrt(jnp.maximum(deg_in, 1.0))


_prep_call = pl.pallas_call(
    _prep_body,
    out_shape=(
        jax.ShapeDtypeStruct((NPAD, D), jnp.float32),
        jax.ShapeDtypeStruct((NPAD, 1), jnp.float32),
    ),
)


def _score_body(a0_ref, a1_ref, ni_ref, w_ref, s_ref):
    agg = (a0_ref[...] + a1_ref[...]) * ni_ref[...]
    s_ref[...] = jnp.dot(agg, w_ref[...], preferred_element_type=jnp.float32)


_score_call = pl.pallas_call(
    _score_body,
    out_shape=jax.ShapeDtypeStruct((NPAD, 1), jnp.float32),
)


def _topk_body(sg_ref, b_ref, perm_ref, scale_ref):
    g = pl.program_id(0)
    s = sg_ref[...].reshape(1, GP) + b_ref[0, 0]
    lane = lax.broadcasted_iota(jnp.int32, (1, GP), 1)
    sp = jnp.where(lane < PER, s, NEG)
    sb = jnp.broadcast_to(sp, (GP, GP))
    ir = lax.broadcasted_iota(jnp.int32, (GP, GP), 0)
    ic = lax.broadcasted_iota(jnp.int32, (GP, GP), 1)
    eye = ir == ic
    s_col = jnp.sum(jnp.where(eye, sb, 0.0), axis=1, keepdims=True)
    term = (sb > s_col) | ((sb == s_col) & (ic < ir))
    cnt = jnp.sum(term.astype(jnp.float32), axis=1, keepdims=True)
    onehot = cnt == ic.astype(jnp.float32)
    sel = jnp.sum(jnp.where(onehot, ir.astype(jnp.float32), 0.0), axis=0,
                  keepdims=True)
    scol_b = jnp.broadcast_to(s_col, (GP, GP))
    ssort = jnp.sum(jnp.where(onehot, scol_b, 0.0), axis=0, keepdims=True)
    perm_ref[...] = (sel.astype(jnp.int32) + g * PER).reshape(1, 1, GP)
    scale_ref[...] = jnp.tanh(ssort).reshape(1, 1, GP)


_topk_call = pl.pallas_call(
    _topk_body,
    grid=(B,),
    in_specs=[
        pl.BlockSpec((1, 1, GP), lambda g: (g, 0, 0)),
        pl.BlockSpec((1, 1), lambda g: (0, 0)),
    ],
    out_specs=(
        pl.BlockSpec((1, 1, GP), lambda g: (g, 0, 0)),
        pl.BlockSpec((1, 1, GP), lambda g: (g, 0, 0)),
    ),
    out_shape=(
        jax.ShapeDtypeStruct((B, 1, GP), jnp.int32),
        jax.ShapeDtypeStruct((B, 1, GP), jnp.float32),
    ),
)


def _scale_body(rows_ref, sc_ref, out_ref):
    out_ref[...] = rows_ref[...] * sc_ref[...]


_scale_call = pl.pallas_call(
    _scale_body,
    out_shape=jax.ShapeDtypeStruct((B * KP, D), jnp.float32),
)


def kernel(feature, W, b, edge_index, batch_num_nodes):
    src = edge_index[0]
    dst = edge_index[1]

    degp = _deg_call(src, dst)
    fpad = jnp.concatenate(
        [feature, jnp.zeros((NPAD - N, D), jnp.float32)], axis=0)
    h, norm_in = _prep_call(
        fpad,
        degp[0, 0].reshape(NPAD, 1), degp[1, 0].reshape(NPAD, 1),
        degp[0, 1].reshape(NPAD, 1), degp[1, 1].reshape(NPAD, 1))
    accp = _agg_call(src, dst, h)
    score = _score_call(accp[0], accp[1], norm_in, W)

    zpad = jnp.full((B, GP - PER), NEG, jnp.float32)
    sg = jnp.concatenate([score[:N, 0].reshape(B, PER), zpad], axis=1)
    perm, scale = _topk_call(sg.reshape(B, 1, GP), b.reshape(1, 1))
    perm = perm.reshape(B, GP)
    scale = scale.reshape(B, GP)

    idx = perm[:, :KP].reshape(B * KP)
    rows = _gat_call(feature, idx)
    feat = _scale_call(rows, scale[:, :KP].reshape(B * KP, 1))

    feat_out = feat.reshape(B, KP, D)[:, :K].reshape(B * K, D)
    perm_sel = idx.reshape(B, KP)[:, :K].reshape(B * K)
    return feat_out, perm_sel
